# Initial kernel scaffold; baseline (speedup 1.0000x reference)
import jax
import jax.numpy as jnp
from jax.experimental import pallas as pl


def kernel(h, e, d2, W1, b1, ln_g, ln_b, W2, b2, W3, b3, W4, b4):
    raise NotImplementedError("write your pallas kernel here")


# speedup vs baseline: 2.7589x; 2.7589x over previous
import functools

import jax
import jax.numpy as jnp
from jax import lax
from jax.experimental import pallas as pl
from jax.experimental.pallas import tpu as pltpu
from jax.experimental.pallas import tpu_sc as plsc

NC = 2
NS = 16
NW = NC * NS
CHUNK = 128


def _node_pre(h_ref, wa_ref, wb_ref, b1_ref, ha_ref, hb_ref):
    h = h_ref[...]
    ha_ref[...] = jnp.dot(h, wa_ref[...], preferred_element_type=jnp.float32)
    hb_ref[...] = (jnp.dot(h, wb_ref[...], preferred_element_type=jnp.float32)
                   + b1_ref[...])


def _edge_mlp(ga_ref, gb_ref, d2_ref, w1c_ref, lng_ref, lnb_ref, w2_ref,
              b2_ref, m_ref):
    x = ga_ref[...] + gb_ref[...] + d2_ref[...] * w1c_ref[...]
    x = x * jax.nn.sigmoid(x)
    mu = jnp.mean(x, axis=-1, keepdims=True)
    var = jnp.mean((x - mu) ** 2, axis=-1, keepdims=True)
    x = (x - mu) * lax.rsqrt(var + 1e-5) * lng_ref[...] + lnb_ref[...]
    y = jnp.dot(x, w2_ref[...], preferred_element_type=jnp.float32) + b2_ref[...]
    m_ref[...] = y * jax.nn.sigmoid(y)


def _node_upd(h_ref, a0_ref, a1_ref, w3h_ref, w3a_ref, b3_ref, w4_ref, b4_ref,
              o_ref):
    h = h_ref[...]
    agg = a0_ref[...] + a1_ref[...]
    y = (jnp.dot(h, w3h_ref[...], preferred_element_type=jnp.float32)
         + jnp.dot(agg, w3a_ref[...], preferred_element_type=jnp.float32)
         + b3_ref[...])
    y = y * jax.nn.sigmoid(y)
    o_ref[...] = h + jnp.dot(y, w4_ref[...], preferred_element_type=jnp.float32) + b4_ref[...]


def _sc_gather(ha, hb, ei2, ej2, ga, gb, idxa, idxb, bufa, bufb, sema, semb):
    kb = idxa.shape[0]
    wid = lax.axis_index("s") * NC + lax.axis_index("c")
    pltpu.sync_copy(ei2.at[wid], idxa)
    pltpu.sync_copy(ej2.at[wid], idxb)

    def body(j, carry):
        row0 = (wid * kb + j) * CHUNK
        ca = pltpu.async_copy(ha.at[idxa.at[j]], bufa, sema)
        cb = pltpu.async_copy(hb.at[idxb.at[j]], bufb, semb)
        ca.wait()
        cb.wait()
        pltpu.sync_copy(bufa, ga.at[pl.ds(row0, CHUNK)])
        pltpu.sync_copy(bufb, gb.at[pl.ds(row0, CHUNK)])
        return carry

    lax.fori_loop(0, kb, body, 0)


def _sc_scatter(m, eis2, zrows, out, idx, buf, shared):
    kb = idx.shape[0]
    np_rows = shared.shape[0]
    rpt = np_rows // NS
    cid = lax.axis_index("c")
    sid = lax.axis_index("s")
    w2 = cid * NS + sid

    pltpu.sync_copy(zrows.at[pl.ds(sid * rpt, rpt)],
                    shared.at[pl.ds(sid * rpt, rpt)])
    plsc.subcore_barrier()

    pltpu.sync_copy(eis2.at[w2], idx)

    def body(j, carry):
        row0 = (w2 * kb + j) * CHUNK
        pltpu.sync_copy(m.at[pl.ds(row0, CHUNK)], buf)
        pltpu.sync_copy(buf, shared.at[idx.at[j]], add=True)
        return carry

    lax.fori_loop(0, kb, body, 0)
    plsc.subcore_barrier()

    pltpu.sync_copy(shared.at[pl.ds(sid * rpt, rpt)],
                    out.at[pl.ds(cid * np_rows + sid * rpt, rpt)])


def _ceil_to(x, m):
    return (x + m - 1) // m * m


def kernel(h, e, d2, W1, b1, ln_g, ln_b, W2, b2, W3, b3, W4, b4):
    n, d = h.shape
    num_e = e.shape[1]
    ei = e[0]
    ej = e[1]

    kb = -(-num_e // (NW * CHUNK))
    ep = NW * kb * CHUNK
    pad = ep - num_e
    np_rows = _ceil_to(n + 1, NS * 8)

    f32 = jnp.float32
    i32 = jnp.int32

    bn = 2000
    grid0 = n // bn
    hA, hB = pl.pallas_call(
        _node_pre,
        grid=(grid0,),
        in_specs=[
            pl.BlockSpec((bn, d), lambda i: (i, 0)),
            pl.BlockSpec((d, d), lambda i: (0, 0)),
            pl.BlockSpec((d, d), lambda i: (0, 0)),
            pl.BlockSpec((1, d), lambda i: (0, 0)),
        ],
        out_specs=[pl.BlockSpec((bn, d), lambda i: (i, 0)),
                   pl.BlockSpec((bn, d), lambda i: (i, 0))],
        out_shape=[jax.ShapeDtypeStruct((n, d), f32)] * 2,
        compiler_params=pltpu.CompilerParams(
            dimension_semantics=("parallel",)),
    )(h, W1[:d], W1[d:2 * d], b1.reshape(1, d))

    ei2 = jnp.concatenate([ei, jnp.zeros((pad,), i32)]).reshape(NW, kb, CHUNK)
    ej2 = jnp.concatenate([ej, jnp.zeros((pad,), i32)]).reshape(NW, kb, CHUNK)
    mesh = plsc.VectorSubcoreMesh(core_axis_name="c", subcore_axis_name="s",
                                  num_cores=NC, num_subcores=NS)
    gA, gB = pl.kernel(
        _sc_gather,
        out_type=[jax.ShapeDtypeStruct((ep, d), f32)] * 2,
        mesh=mesh,
        scratch_types=[
            pltpu.VMEM((kb, CHUNK), i32),
            pltpu.VMEM((kb, CHUNK), i32),
            pltpu.VMEM((CHUNK, d), f32),
            pltpu.VMEM((CHUNK, d), f32),
            pltpu.SemaphoreType.DMA,
            pltpu.SemaphoreType.DMA,
        ],
    )(hA, hB, ei2, ej2)

    be = 2048
    grid2 = ep // be
    d2p = jnp.concatenate([d2, jnp.zeros((pad, 1), f32)])
    m = pl.pallas_call(
        _edge_mlp,
        grid=(grid2,),
        in_specs=[
            pl.BlockSpec((be, d), lambda i: (i, 0)),
            pl.BlockSpec((be, d), lambda i: (i, 0)),
            pl.BlockSpec((be, 1), lambda i: (i, 0)),
            pl.BlockSpec((1, d), lambda i: (0, 0)),
            pl.BlockSpec((1, d), lambda i: (0, 0)),
            pl.BlockSpec((1, d), lambda i: (0, 0)),
            pl.BlockSpec((d, d), lambda i: (0, 0)),
            pl.BlockSpec((1, d), lambda i: (0, 0)),
        ],
        out_specs=pl.BlockSpec((be, d), lambda i: (i, 0)),
        out_shape=jax.ShapeDtypeStruct((ep, d), f32),
        compiler_params=pltpu.CompilerParams(
            dimension_semantics=("parallel",)),
    )(gA, gB, d2p, W1[2 * d].reshape(1, d), ln_g.reshape(1, d),
      ln_b.reshape(1, d), W2, b2.reshape(1, d))

    ei_s = jnp.concatenate([ei, jnp.full((pad,), n, i32)]).reshape(NW, kb, CHUNK)
    zrows = jnp.zeros((np_rows, d), f32)
    agg2 = pl.kernel(
        _sc_scatter,
        out_type=jax.ShapeDtypeStruct((NC * np_rows, d), f32),
        mesh=mesh,
        scratch_types=[
            pltpu.VMEM((kb, CHUNK), i32),
            pltpu.VMEM((CHUNK, d), f32),
            pltpu.VMEM_SHARED((np_rows, d), f32),
        ],
    )(m, ei_s, zrows)

    out = pl.pallas_call(
        _node_upd,
        grid=(grid0,),
        in_specs=[
            pl.BlockSpec((bn, d), lambda i: (i, 0)),
            pl.BlockSpec((bn, d), lambda i: (i, 0)),
            pl.BlockSpec((bn, d), lambda i: (i, 0)),
            pl.BlockSpec((d, d), lambda i: (0, 0)),
            pl.BlockSpec((d, d), lambda i: (0, 0)),
            pl.BlockSpec((1, d), lambda i: (0, 0)),
            pl.BlockSpec((d, d), lambda i: (0, 0)),
            pl.BlockSpec((1, d), lambda i: (0, 0)),
        ],
        out_specs=pl.BlockSpec((bn, d), lambda i: (i, 0)),
        out_shape=jax.ShapeDtypeStruct((n, d), f32),
        compiler_params=pltpu.CompilerParams(
            dimension_semantics=("parallel",)),
    )(h, agg2[:n], agg2[np_rows:np_rows + n], W3[:d], W3[d:],
      b3.reshape(1, d), W4, b4.reshape(1, d))
    return out


# speedup vs baseline: 3.0398x; 1.1018x over previous
import functools

import jax
import jax.numpy as jnp
from jax import lax
from jax.experimental import pallas as pl
from jax.experimental.pallas import tpu as pltpu
from jax.experimental.pallas import tpu_sc as plsc

NC = 2
NS = 16
NW = NC * NS
CHUNK = 128


def _node_pre(h_ref, wa_ref, wb_ref, b1_ref, ha_ref, hb_ref):
    h = h_ref[...]
    ha_ref[...] = jnp.dot(h, wa_ref[...], preferred_element_type=jnp.float32)
    hb_ref[...] = (jnp.dot(h, wb_ref[...], preferred_element_type=jnp.float32)
                   + b1_ref[...])


def _edge_mlp(ga_ref, gb_ref, d2_ref, w1c_ref, lng_ref, lnb_ref, w2_ref,
              b2_ref, m_ref):
    x = ga_ref[...] + gb_ref[...] + d2_ref[...] * w1c_ref[...]
    x = x * jax.nn.sigmoid(x)
    mu = jnp.mean(x, axis=-1, keepdims=True)
    var = jnp.mean((x - mu) ** 2, axis=-1, keepdims=True)
    x = (x - mu) * lax.rsqrt(var + 1e-5) * lng_ref[...] + lnb_ref[...]
    y = jnp.dot(x, w2_ref[...], preferred_element_type=jnp.float32) + b2_ref[...]
    m_ref[...] = y * jax.nn.sigmoid(y)


def _node_upd(h_ref, a0_ref, a1_ref, w3h_ref, w3a_ref, b3_ref, w4_ref, b4_ref,
              o_ref):
    h = h_ref[...]
    agg = a0_ref[...] + a1_ref[...]
    y = (jnp.dot(h, w3h_ref[...], preferred_element_type=jnp.float32)
         + jnp.dot(agg, w3a_ref[...], preferred_element_type=jnp.float32)
         + b3_ref[...])
    y = y * jax.nn.sigmoid(y)
    o_ref[...] = h + jnp.dot(y, w4_ref[...], preferred_element_type=jnp.float32) + b4_ref[...]


def _sc_gather(ha, hb, ei2, ej2, ga, gb, idxa, idxb,
               ba0, ba1, bb0, bb1, sg0, sg1, sw0, sw1):
    kb = idxa.shape[0]
    wid = lax.axis_index("s") * NC + lax.axis_index("c")
    pltpu.sync_copy(ei2.at[wid], idxa)
    pltpu.sync_copy(ej2.at[wid], idxb)
    bas, bbs, sgs, sws = (ba0, ba1), (bb0, bb1), (sg0, sg1), (sw0, sw1)

    def wait_gather(b):
        pltpu.make_async_copy(ha.at[pl.ds(0, CHUNK)], bas[b], sgs[b]).wait()
        pltpu.make_async_copy(hb.at[pl.ds(0, CHUNK)], bbs[b], sgs[b]).wait()

    def wait_write(b):
        pltpu.make_async_copy(ha.at[pl.ds(0, CHUNK)], bas[b], sws[b]).wait()
        pltpu.make_async_copy(hb.at[pl.ds(0, CHUNK)], bbs[b], sws[b]).wait()

    pltpu.async_copy(ha.at[idxa.at[0]], ba0, sg0)
    pltpu.async_copy(hb.at[idxb.at[0]], bb0, sg0)

    def step(b, other):
        def body(j, carry):
            @pl.when(j + 1 < kb)
            def _():
                @pl.when(j >= 1)
                def _():
                    wait_write(other)
                pltpu.async_copy(ha.at[idxa.at[j + 1]], bas[other], sgs[other])
                pltpu.async_copy(hb.at[idxb.at[j + 1]], bbs[other], sgs[other])
            wait_gather(b)
            row0 = (wid * kb + j) * CHUNK
            pltpu.async_copy(bas[b], ga.at[pl.ds(row0, CHUNK)], sws[b])
            pltpu.async_copy(bbs[b], gb.at[pl.ds(row0, CHUNK)], sws[b])
            return carry
        return body

    def pair(p, carry):
        carry = step(0, 1)(2 * p, carry)
        carry = step(1, 0)(2 * p + 1, carry)
        return carry

    carry = lax.fori_loop(0, kb // 2, pair, 0)
    if kb % 2:
        step(0, 1)(kb - 1, carry)
    wait_write(0)
    if kb > 1:
        wait_write(1)


def _sc_scatter(m, eis2, zrows, out, idx, b0, b1, sr0, sr1, sa0, sa1, shared):
    kb = idx.shape[0]
    np_rows = shared.shape[0]
    rpt = np_rows // NS
    cid = lax.axis_index("c")
    sid = lax.axis_index("s")
    w2 = cid * NS + sid
    bufs, srs, sas = (b0, b1), (sr0, sr1), (sa0, sa1)

    pltpu.sync_copy(eis2.at[w2], idx)
    pltpu.async_copy(m.at[pl.ds(w2 * kb * CHUNK, CHUNK)], b0, sr0)

    pltpu.sync_copy(zrows.at[pl.ds(sid * rpt, rpt)],
                    shared.at[pl.ds(sid * rpt, rpt)])
    plsc.subcore_barrier()

    def wait_read(b):
        pltpu.make_async_copy(m.at[pl.ds(0, CHUNK)], bufs[b], srs[b]).wait()

    def wait_add(b):
        pltpu.make_async_copy(m.at[pl.ds(0, CHUNK)], bufs[b], sas[b]).wait()

    def step(b, other):
        def body(j, carry):
            @pl.when(j + 1 < kb)
            def _():
                @pl.when(j >= 1)
                def _():
                    wait_add(other)
                row1 = (w2 * kb + j + 1) * CHUNK
                pltpu.async_copy(m.at[pl.ds(row1, CHUNK)], bufs[other],
                                 srs[other])
            wait_read(b)
            pltpu.async_copy(bufs[b], shared.at[idx.at[j]], sas[b], add=True)
            return carry
        return body

    def pair(p, carry):
        carry = step(0, 1)(2 * p, carry)
        carry = step(1, 0)(2 * p + 1, carry)
        return carry

    carry = lax.fori_loop(0, kb // 2, pair, 0)
    if kb % 2:
        step(0, 1)(kb - 1, carry)
    wait_add(0)
    if kb > 1:
        wait_add(1)
    plsc.subcore_barrier()

    pltpu.sync_copy(shared.at[pl.ds(sid * rpt, rpt)],
                    out.at[pl.ds(cid * np_rows + sid * rpt, rpt)])


def _ceil_to(x, m):
    return (x + m - 1) // m * m


def kernel(h, e, d2, W1, b1, ln_g, ln_b, W2, b2, W3, b3, W4, b4):
    n, d = h.shape
    num_e = e.shape[1]
    ei = e[0]
    ej = e[1]

    kb = -(-num_e // (NW * CHUNK))
    ep = NW * kb * CHUNK
    pad = ep - num_e
    np_rows = _ceil_to(n + 1, NS * 8)

    f32 = jnp.float32
    i32 = jnp.int32

    bn = 2000
    grid0 = n // bn
    hA, hB = pl.pallas_call(
        _node_pre,
        grid=(grid0,),
        in_specs=[
            pl.BlockSpec((bn, d), lambda i: (i, 0)),
            pl.BlockSpec((d, d), lambda i: (0, 0)),
            pl.BlockSpec((d, d), lambda i: (0, 0)),
            pl.BlockSpec((1, d), lambda i: (0, 0)),
        ],
        out_specs=[pl.BlockSpec((bn, d), lambda i: (i, 0)),
                   pl.BlockSpec((bn, d), lambda i: (i, 0))],
        out_shape=[jax.ShapeDtypeStruct((n, d), f32)] * 2,
        compiler_params=pltpu.CompilerParams(
            dimension_semantics=("parallel",)),
    )(h, W1[:d], W1[d:2 * d], b1.reshape(1, d))

    ei2 = jnp.concatenate([ei, jnp.zeros((pad,), i32)]).reshape(NW, kb, CHUNK)
    ej2 = jnp.concatenate([ej, jnp.zeros((pad,), i32)]).reshape(NW, kb, CHUNK)
    mesh = plsc.VectorSubcoreMesh(core_axis_name="c", subcore_axis_name="s",
                                  num_cores=NC, num_subcores=NS)
    gA, gB = pl.kernel(
        _sc_gather,
        out_type=[jax.ShapeDtypeStruct((ep, d), f32)] * 2,
        mesh=mesh,
        scratch_types=[
            pltpu.VMEM((kb, CHUNK), i32),
            pltpu.VMEM((kb, CHUNK), i32),
            pltpu.VMEM((CHUNK, d), f32),
            pltpu.VMEM((CHUNK, d), f32),
            pltpu.VMEM((CHUNK, d), f32),
            pltpu.VMEM((CHUNK, d), f32),
            pltpu.SemaphoreType.DMA,
            pltpu.SemaphoreType.DMA,
            pltpu.SemaphoreType.DMA,
            pltpu.SemaphoreType.DMA,
        ],
    )(hA, hB, ei2, ej2)

    be = 2048
    grid2 = ep // be
    d2p = jnp.concatenate([d2, jnp.zeros((pad, 1), f32)])
    m = pl.pallas_call(
        _edge_mlp,
        grid=(grid2,),
        in_specs=[
            pl.BlockSpec((be, d), lambda i: (i, 0)),
            pl.BlockSpec((be, d), lambda i: (i, 0)),
            pl.BlockSpec((be, 1), lambda i: (i, 0)),
            pl.BlockSpec((1, d), lambda i: (0, 0)),
            pl.BlockSpec((1, d), lambda i: (0, 0)),
            pl.BlockSpec((1, d), lambda i: (0, 0)),
            pl.BlockSpec((d, d), lambda i: (0, 0)),
            pl.BlockSpec((1, d), lambda i: (0, 0)),
        ],
        out_specs=pl.BlockSpec((be, d), lambda i: (i, 0)),
        out_shape=jax.ShapeDtypeStruct((ep, d), f32),
        compiler_params=pltpu.CompilerParams(
            dimension_semantics=("parallel",)),
    )(gA, gB, d2p, W1[2 * d].reshape(1, d), ln_g.reshape(1, d),
      ln_b.reshape(1, d), W2, b2.reshape(1, d))

    ei_s = jnp.concatenate([ei, jnp.full((pad,), n, i32)]).reshape(NW, kb, CHUNK)
    zrows = jnp.zeros((np_rows, d), f32)
    agg2 = pl.kernel(
        _sc_scatter,
        out_type=jax.ShapeDtypeStruct((NC * np_rows, d), f32),
        mesh=mesh,
        scratch_types=[
            pltpu.VMEM((kb, CHUNK), i32),
            pltpu.VMEM((CHUNK, d), f32),
            pltpu.VMEM((CHUNK, d), f32),
            pltpu.SemaphoreType.DMA,
            pltpu.SemaphoreType.DMA,
            pltpu.SemaphoreType.DMA,
            pltpu.SemaphoreType.DMA,
            pltpu.VMEM_SHARED((np_rows, d), f32),
        ],
    )(m, ei_s, zrows)

    out = pl.pallas_call(
        _node_upd,
        grid=(grid0,),
        in_specs=[
            pl.BlockSpec((bn, d), lambda i: (i, 0)),
            pl.BlockSpec((bn, d), lambda i: (i, 0)),
            pl.BlockSpec((bn, d), lambda i: (i, 0)),
            pl.BlockSpec((d, d), lambda i: (0, 0)),
            pl.BlockSpec((d, d), lambda i: (0, 0)),
            pl.BlockSpec((1, d), lambda i: (0, 0)),
            pl.BlockSpec((d, d), lambda i: (0, 0)),
            pl.BlockSpec((1, d), lambda i: (0, 0)),
        ],
        out_specs=pl.BlockSpec((bn, d), lambda i: (i, 0)),
        out_shape=jax.ShapeDtypeStruct((n, d), f32),
        compiler_params=pltpu.CompilerParams(
            dimension_semantics=("parallel",)),
    )(h, agg2[:n], agg2[np_rows:np_rows + n], W3[:d], W3[d:],
      b3.reshape(1, d), W4, b4.reshape(1, d))
    return out
